# baseline (speedup 1.0000x reference)
import functools

import jax
import jax.numpy as jnp
from jax import lax
from jax.experimental import pallas as pl
from jax.experimental.pallas import tpu as pltpu
from jax.experimental.pallas import tpu_sc as plsc

_ALPHA_LN = 2.302585092994046

_B, _V, _S, _P, _K = 64, 16, 512, 64, 8
_NC, _NS = 2, 16
_NW = _NC * _NS
_NROWS = _B * _V
_ROWS_PER_W = _NROWS // _NW
_OPAD = _S * _K + 8


def _sc_body(x_hbm, c_hbm, i_hbm, o_hbm, xrow, orow, cbuf, ibuf):
    wid = lax.axis_index("s") * _NC + lax.axis_index("c")
    pltpu.sync_copy(c_hbm, cbuf)
    pltpu.sync_copy(i_hbm, ibuf)
    iota = lax.iota(jnp.int32, 16)
    lo8 = iota < 8

    def row_body(r, carry):
        row = wid * _ROWS_PER_W + r
        v = lax.rem(row, _V)
        pltpu.sync_copy(x_hbm.at[row], xrow)
        cvecs = [cbuf[pl.ds(v * _P + 16 * q, 16)] for q in range(4)]
        ivecs = [ibuf[pl.ds(v * _P + 16 * q, 16)] for q in range(4)]

        def scal_body(j, carry2):
            xc = xrow[pl.ds(j * 16, 16)]
            for l in range(16):
                xv = lax.gather(
                    xc, jnp.full((16, 1), l, jnp.int32),
                    lax.GatherDimensionNumbers(
                        offset_dims=(), collapsed_slice_dims=(0,),
                        start_index_map=(0,)),
                    slice_sizes=(1,),
                    mode=lax.GatherScatterMode.PROMISE_IN_BOUNDS)
                def _srt(a):
                    ks, _vs = plsc.sort_key_val(a, a)
                    return ks

                ss = []
                for q in range(4):
                    d = xv - cvecs[q]
                    ss.append(_srt((d * d) * ivecs[q]))
                m1 = _srt(jnp.where(lo8, ss[0], lax.rev(ss[1], (0,))))
                m2 = _srt(jnp.where(lo8, ss[2], lax.rev(ss[3], (0,))))
                m3 = _srt(jnp.where(lo8, m1, lax.rev(m2, (0,))))
                mv = jnp.where(m3 <= _ALPHA_LN, jnp.exp(-m3),
                               jnp.zeros((16,), jnp.float32))
                plsc.store_compressed(orow.at[pl.ds(j * 128 + 8 * l, 16)], mv,
                                      mask=lo8)
            return carry2

        lax.fori_loop(0, _S // 16, scal_body, 0)
        pltpu.sync_copy(orow.at[pl.ds(0, _S * _K)], o_hbm.at[row])
        return carry

    lax.fori_loop(0, _ROWS_PER_W, row_body, 0)


def kernel(x, fuzzy_sets, k):
    B, V, S = x.shape
    c = fuzzy_sets[:, :, 0].reshape(-1)
    sig = fuzzy_sets[:, :, 1]
    inv = (1.0 / (2.0 * sig * sig)).reshape(-1)
    xrows = x.reshape(B * V, S)
    fn = pl.kernel(
        _sc_body,
        out_type=jax.ShapeDtypeStruct((_NROWS, _S * _K), jnp.float32),
        mesh=plsc.VectorSubcoreMesh(core_axis_name="c", subcore_axis_name="s"),
        compiler_params=pltpu.CompilerParams(use_tc_tiling_on_sc=False, needs_layout_passes=False),
        scratch_types=[
            pltpu.VMEM((_S,), jnp.float32),
            pltpu.VMEM((_OPAD,), jnp.float32),
            pltpu.VMEM((_V * _P,), jnp.float32),
            pltpu.VMEM((_V * _P,), jnp.float32),
        ],
    )
    out = fn(xrows, c, inv)
    return out.reshape(B, V, S, _K)


# speedup vs baseline: 3.6299x; 3.6299x over previous
import jax
import jax.numpy as jnp
from jax import lax
from jax.experimental import pallas as pl
from jax.experimental.pallas import tpu as pltpu
from jax.experimental.pallas import tpu_sc as plsc

_ALPHA_LN = 2.302585092994046

_V, _S, _P, _K = 16, 512, 64, 8
_VB = 2
_SC_B = 8
_NC, _NS = 2, 16
_NW = _NC * _NS


_SORT8_NET = (
    (0, 1), (2, 3), (4, 5), (6, 7),
    (0, 2), (1, 3), (4, 6), (5, 7),
    (1, 2), (5, 6),
    (0, 4), (1, 5), (2, 6), (3, 7),
    (2, 4), (3, 5),
    (1, 2), (3, 4), (5, 6),
)


def _ce(lst, i, j):
    a, b = lst[i], lst[j]
    lst[i] = jnp.minimum(a, b)
    lst[j] = jnp.maximum(a, b)


def _sort8(vals):
    lst = list(vals)
    for i, j in _SORT8_NET:
        _ce(lst, i, j)
    return lst


def _merge_bottom8(A, B):
    C = [jnp.minimum(A[i], B[7 - i]) for i in range(8)]
    for d in (4, 2, 1):
        for i in range(8):
            if (i & d) == 0 and (i | d) < 8:
                _ce(C, i, i + d)
    return C


def _tc_body(c_ref, inv_ref, x_ref, o_ref):
    for vv in range(_VB):
        xb = x_ref[vv]
        R = None
        for g in range(_P // 8):
            grp = []
            for t in range(8):
                p = g * 8 + t
                d = xb - c_ref[vv, 0, p].astype(jnp.bfloat16)
                grp.append((d * d) * inv_ref[vv, 0, p].astype(jnp.bfloat16))
            grp = _sort8(grp)
            R = grp if R is None else _merge_bottom8(R, grp)
        for j in range(_K):
            u = R[j].astype(jnp.float32)
            o_ref[:, vv, j, :] = jnp.where(u <= _ALPHA_LN, jnp.exp(-u), 0.0)


def _tc_part(x, c, inv, cb):
    grid = (1, _V // _VB, 1)
    out = pl.pallas_call(
        _tc_body,
        grid=grid,
        in_specs=[
            pl.BlockSpec((_VB, 1, _P), lambda i, j, s: (j, 0, 0),
                         memory_space=pltpu.SMEM),
            pl.BlockSpec((_VB, 1, _P), lambda i, j, s: (j, 0, 0),
                         memory_space=pltpu.SMEM),
            pl.BlockSpec((_VB, cb, _S), lambda i, j, s: (j, i, s)),
        ],
        out_specs=pl.BlockSpec((cb, _VB, _K, _S), lambda i, j, s: (i, j, 0, s)),
        out_shape=jax.ShapeDtypeStruct((cb, _V, _K, _S), jnp.float32),
        compiler_params=pltpu.CompilerParams(
            dimension_semantics=("parallel", "parallel", "parallel")),
    )(c.reshape(_V, 1, _P), inv.reshape(_V, 1, _P), x.astype(jnp.bfloat16))
    return jnp.transpose(out, (0, 1, 3, 2))


_SC_ROWS = _SC_B * _V
_SC_RPW = _SC_ROWS // _NW
_OPAD = _S * _K + 8


def _sc_body(x_hbm, c_hbm, i_hbm, o_hbm, xrow, orow, cbuf, ibuf):
    wid = lax.axis_index("s") * _NC + lax.axis_index("c")
    pltpu.sync_copy(c_hbm, cbuf)
    pltpu.sync_copy(i_hbm, ibuf)
    iota = lax.iota(jnp.int32, 16)
    lo8 = iota < 8

    def row_body(r, carry):
        row = wid * _SC_RPW + r
        v = lax.rem(row, _V)
        pltpu.sync_copy(x_hbm.at[row], xrow)
        cvecs = [cbuf[pl.ds(v * _P + 16 * q, 16)] for q in range(4)]
        ivecs = [ibuf[pl.ds(v * _P + 16 * q, 16)] for q in range(4)]

        def scal_body(j, carry2):
            xc = xrow[pl.ds(j * 16, 16)]
            for l in range(16):
                xv = lax.gather(
                    xc, jnp.full((16, 1), l, jnp.int32),
                    lax.GatherDimensionNumbers(
                        offset_dims=(), collapsed_slice_dims=(0,),
                        start_index_map=(0,)),
                    slice_sizes=(1,),
                    mode=lax.GatherScatterMode.PROMISE_IN_BOUNDS)

                def _srt(a):
                    ks, _vs = plsc.sort_key_val(a, a)
                    return ks

                ss = []
                for q in range(4):
                    d = xv - cvecs[q]
                    ss.append(_srt((d * d) * ivecs[q]))
                m1 = _srt(jnp.where(lo8, ss[0], lax.rev(ss[1], (0,))))
                m2 = _srt(jnp.where(lo8, ss[2], lax.rev(ss[3], (0,))))
                m3 = _srt(jnp.where(lo8, m1, lax.rev(m2, (0,))))
                mv = jnp.where(m3 <= _ALPHA_LN, jnp.exp(-m3),
                               jnp.zeros((16,), jnp.float32))
                plsc.store_compressed(orow.at[pl.ds(j * 128 + 8 * l, 16)], mv,
                                      mask=lo8)
            return carry2

        lax.fori_loop(0, _S // 16, scal_body, 0)
        pltpu.sync_copy(orow.at[pl.ds(0, _S * _K)], o_hbm.at[row])
        return carry

    lax.fori_loop(0, _SC_RPW, row_body, 0)


def _sc_part(xrows, c, inv):
    fn = pl.kernel(
        _sc_body,
        out_type=jax.ShapeDtypeStruct((_SC_ROWS, _S * _K), jnp.float32),
        mesh=plsc.VectorSubcoreMesh(core_axis_name="c", subcore_axis_name="s"),
        compiler_params=pltpu.CompilerParams(
            use_tc_tiling_on_sc=False, needs_layout_passes=False),
        scratch_types=[
            pltpu.VMEM((_S,), jnp.float32),
            pltpu.VMEM((_OPAD,), jnp.float32),
            pltpu.VMEM((_V * _P,), jnp.float32),
            pltpu.VMEM((_V * _P,), jnp.float32),
        ],
    )
    return fn(xrows, c, inv)


def kernel(x, fuzzy_sets, k):
    B, V, S = x.shape
    c = fuzzy_sets[:, :, 0].reshape(-1)
    sig = fuzzy_sets[:, :, 1]
    inv = (1.0 / (2.0 * sig * sig)).reshape(-1)

    tc_b = B - _SC_B
    x_tc = jnp.transpose(x[:tc_b], (1, 0, 2))
    x_sc = x[tc_b:].reshape(_SC_ROWS, S)

    out_sc = _sc_part(x_sc, c, inv)
    out_tc = _tc_part(x_tc, c, inv, tc_b)
    out_sc = out_sc.reshape(_SC_B, V, S, _K)
    return jnp.concatenate([out_tc, out_sc], axis=0)


# speedup vs baseline: 8.0920x; 2.2292x over previous
import jax
import jax.numpy as jnp
from jax.experimental import pallas as pl
from jax.experimental.pallas import tpu as pltpu

_ALPHA_LN = 2.302585092994046

_CB = 64
_VB = 2
_SCHUNK = 512
_P = 64
_K = 8

_SORT8_NET = (
    (0, 1), (2, 3), (4, 5), (6, 7),
    (0, 2), (1, 3), (4, 6), (5, 7),
    (1, 2), (5, 6),
    (0, 4), (1, 5), (2, 6), (3, 7),
    (2, 4), (3, 5),
    (1, 2), (3, 4), (5, 6),
)


def _ce(lst, i, j):
    a, b = lst[i], lst[j]
    lst[i] = jnp.minimum(a, b)
    lst[j] = jnp.maximum(a, b)


def _sort8(vals):
    lst = list(vals)
    for i, j in _SORT8_NET:
        _ce(lst, i, j)
    return lst


def _merge_bottom8(A, B):
    C = [jnp.minimum(A[i], B[7 - i]) for i in range(8)]
    for d in (4, 2, 1):
        for i in range(8):
            if (i & d) == 0 and (i | d) < 8:
                _ce(C, i, i + d)
    return C


def _body(c_ref, inv_ref, x_ref, o_ref):
    for vv in range(_VB):
        xb = x_ref[vv]
        R = None
        for g in range(_P // 8):
            grp = []
            for t in range(8):
                p = g * 8 + t
                d = xb - c_ref[vv, 0, p]
                grp.append((d * d) * inv_ref[vv, 0, p])
            grp = _sort8(grp)
            R = grp if R is None else _merge_bottom8(R, grp)
        for j in range(_K):
            u = R[j].astype(jnp.float32)
            o_ref[:, vv, j, :] = jnp.where(u <= _ALPHA_LN, jnp.exp(-u), 0.0)


def kernel(x, fuzzy_sets, k):
    B, V, S = x.shape
    c = fuzzy_sets[:, :, 0].reshape(V, 1, _P).astype(jnp.bfloat16)
    sig = fuzzy_sets[:, :, 1]
    inv = (1.0 / (2.0 * sig * sig)).reshape(V, 1, _P).astype(jnp.bfloat16)
    grid = (B // _CB, V // _VB, S // _SCHUNK)
    out = pl.pallas_call(
        _body,
        grid=grid,
        in_specs=[
            pl.BlockSpec((_VB, 1, _P), lambda i, j, s: (j, 0, 0), memory_space=pltpu.SMEM),
            pl.BlockSpec((_VB, 1, _P), lambda i, j, s: (j, 0, 0), memory_space=pltpu.SMEM),
            pl.BlockSpec((_VB, _CB, _SCHUNK), lambda i, j, s: (j, i, s)),
        ],
        out_specs=pl.BlockSpec((_CB, _VB, _K, _SCHUNK), lambda i, j, s: (i, j, 0, s)),
        out_shape=jax.ShapeDtypeStruct((B, V, _K, S), jnp.float32),
        compiler_params=pltpu.CompilerParams(
            dimension_semantics=("parallel", "parallel", "parallel")),
    )(c, inv, jnp.transpose(x, (1, 0, 2)).astype(jnp.bfloat16))
    return jnp.transpose(out, (0, 1, 3, 2))
